# baseline (speedup 1.0000x reference)
import functools

import jax
import jax.numpy as jnp
from jax import lax
from jax.experimental import pallas as pl
from jax.experimental.pallas import tpu as pltpu
from jax.experimental.pallas import tpu_sc as plsc

NSC = 2
NSUB = 16
CHUNK = 6400
ROWS = 16
RLEN = CHUNK // ROWS
WPS = 4
PAD = 5


@functools.partial(jax.jit, static_argnums=(2, 3, 4))
def _edge_dot(xsl, ei, E, N, D):
    nch = (E // NSC) // CHUNK
    mesh = plsc.VectorSubcoreMesh(core_axis_name="c", subcore_axis_name="s")

    @functools.partial(
        pl.kernel,
        out_type=jax.ShapeDtypeStruct((NSC, nch, CHUNK), jnp.float32),
        mesh=mesh,
        compiler_params=pltpu.CompilerParams(needs_layout_passes=False,
                                             use_tc_tiling_on_sc=False),
        scratch_types=[
            pltpu.VMEM((N * PAD,), jnp.int32),
            pltpu.VMEM((2, CHUNK), jnp.int32),
            pltpu.VMEM((2, CHUNK), jnp.int32),
            pltpu.VMEM((ROWS, RLEN), jnp.float32),
            pltpu.VMEM((ROWS, RLEN), jnp.float32),
            pltpu.VMEM((RLEN,), jnp.float32),
            pltpu.VMEM_SHARED((2, ROWS, RLEN), jnp.float32),
            pltpu.SemaphoreType.DMA,
            pltpu.SemaphoreType.DMA,
            pltpu.SemaphoreType.DMA,
            pltpu.SemaphoreType.DMA,
        ],
    )
    def k(xsl_hbm, ei_hbm, out_hbm, tbl, sidx, didx, part0, part1, zrow,
          sbuf, ss0, sd0, ss1, sd1):
        c = lax.axis_index("c")
        sid = lax.axis_index("s")
        pltpu.sync_copy(xsl_hbm.at[sid], tbl)
        zv = jnp.zeros((16,), jnp.float32)
        for i in range(RLEN // 16):
            zrow[pl.ds(i * 16, 16)] = zv
        pltpu.sync_copy(zrow, sbuf.at[0, sid])
        pltpu.sync_copy(zrow, sbuf.at[1, sid])
        rowids = lax.iota(jnp.int32, 16)

        def start_idx(ch, q, ss, sd):
            pltpu.async_copy(ei_hbm.at[0, c, ch], sidx.at[q], ss)
            pltpu.async_copy(ei_hbm.at[1, c, ch], didx.at[q], sd)

        def wait_idx(ch, q, ss, sd):
            pltpu.make_async_copy(ei_hbm.at[0, c, ch], sidx.at[q], ss).wait()
            pltpu.make_async_copy(ei_hbm.at[1, c, ch], didx.at[q], sd).wait()

        def chunk_work(ch, q, part):
            @plsc.parallel_loop(0, ROWS, step=1, unroll=1)
            def row_body(r):
                for gg in range(RLEN // 16):
                    off = pl.multiple_of(r * CHUNK // ROWS + gg * 16, 16)
                    s16 = sidx[q, pl.ds(off, 16)]
                    d16 = didx[q, pl.ds(off, 16)]
                    sb = s16 * PAD
                    db = d16 * PAD
                    acc = None
                    for w in range(WPS):
                        sw = sb + w if w else sb
                        dw = db + w if w else db
                        sv = plsc.bitcast(plsc.load_gather(tbl, [sw]),
                                          jnp.bfloat16)
                        dv = plsc.bitcast(plsc.load_gather(tbl, [dw]),
                                          jnp.bfloat16)
                        sa, sb2 = plsc.unpack(
                            sv, format=plsc.PackFormat.INTERLEAVED)
                        da, db2 = plsc.unpack(
                            dv, format=plsc.PackFormat.INTERLEAVED)
                        pr = sa * da + sb2 * db2
                        acc = pr if acc is None else acc + pr
                    part[r, pl.ds(gg * 16, 16)] = acc

            pltpu.sync_copy(part, sbuf.at[q].at[rowids], add=True)
            plsc.subcore_barrier()
            pltpu.sync_copy(sbuf.at[q, sid],
                            out_hbm.at[c, ch, pl.ds(
                                pl.multiple_of(sid * RLEN, 8), RLEN)])
            pltpu.sync_copy(zrow, sbuf.at[q, sid])

        plsc.subcore_barrier()
        start_idx(0, 0, ss0, sd0)

        def body(i, carry):
            ch = i * 2
            start_idx(ch + 1, 1, ss1, sd1)
            wait_idx(ch, 0, ss0, sd0)
            chunk_work(ch, 0, part0)

            @pl.when(ch + 2 < nch)
            def _():
                start_idx(ch + 2, 0, ss0, sd0)

            wait_idx(ch + 1, 1, ss1, sd1)
            chunk_work(ch + 1, 1, part1)
            return carry

        lax.fori_loop(0, (nch - 1) // 2, body, 0)
        wait_idx(nch - 1, 0, ss0, sd0)
        chunk_work(nch - 1, 0, part0)

    return k(xsl, ei)


def kernel(x, edge_index, W1_l, b1_l, W1_r, W2_l, b2_l, W2_r):
    del W1_l, b1_l, W1_r, W2_l, b2_l, W2_r
    N, D = x.shape
    E = edge_index.shape[1]
    nch = (E // NSC) // CHUNK
    ei = edge_index.astype(jnp.int32).reshape(2, NSC, nch, CHUNK)
    xi = jax.lax.bitcast_convert_type(
        x.astype(jnp.bfloat16).reshape(N, D // 2, 2), jnp.int32)
    xsl = xi.reshape(N, NSUB, WPS).transpose(1, 0, 2)
    xsl = jnp.concatenate(
        [xsl, jnp.zeros((NSUB, N, PAD - WPS), jnp.int32)], axis=-1)
    xsl = xsl.reshape(NSUB, N * PAD)
    out = _edge_dot(xsl, ei, E, N, D)
    return out.reshape(E)


# speedup vs baseline: 2.2675x; 2.2675x over previous
import functools

import jax
import jax.numpy as jnp
from jax import lax
from jax.experimental import pallas as pl
from jax.experimental.pallas import tpu as pltpu
from jax.experimental.pallas import tpu_sc as plsc

NW = 32
BLK = 80


@functools.partial(jax.jit, static_argnums=(2, 3, 4))
def _edge_dot(x, ei, E, N, D):
    epw = E // NW
    nb = epw // BLK
    mesh = plsc.VectorSubcoreMesh(core_axis_name="c", subcore_axis_name="s")

    @functools.partial(
        pl.kernel,
        out_type=jax.ShapeDtypeStruct((NW, epw), jnp.float32),
        mesh=mesh,
        compiler_params=pltpu.CompilerParams(needs_layout_passes=False,
                                             use_tc_tiling_on_sc=False),
        scratch_types=[
            pltpu.VMEM((nb, BLK), jnp.int32),
            pltpu.VMEM((nb, BLK), jnp.int32),
            pltpu.VMEM((BLK, D // 2), jnp.int32),
            pltpu.VMEM((BLK, D // 2), jnp.int32),
            pltpu.VMEM((BLK, D // 2), jnp.int32),
            pltpu.VMEM((BLK, D // 2), jnp.int32),
            pltpu.VMEM((BLK, D // 2), jnp.int32),
            pltpu.VMEM((BLK, D // 2), jnp.int32),
            pltpu.VMEM((BLK, D // 2), jnp.int32),
            pltpu.VMEM((BLK, D // 2), jnp.int32),
            pltpu.VMEM((epw,), jnp.float32),
            pltpu.VMEM((BLK * 16,), jnp.float32),
        ] + [pltpu.SemaphoreType.DMA] * 8,
    )
    def k(x_hbm, ei_hbm, out_hbm, sidx, didx, sr0, tr0, sr1, tr1, sr2, tr2,
          sr3, tr3, outa, red,
          ss0, sd0, ss1, sd1, ss2, sd2, ss3, sd3):
        wid = lax.axis_index("s") * 2 + lax.axis_index("c")
        pltpu.sync_copy(ei_hbm.at[0, wid], sidx)
        pltpu.sync_copy(ei_hbm.at[1, wid], didx)
        rowbase = lax.iota(jnp.int32, 16) * 16

        def start(b, sr, tr, ss, sd):
            pltpu.async_copy(x_hbm.at[sidx.at[b]], sr, ss)
            pltpu.async_copy(x_hbm.at[didx.at[b]], tr, sd)

        def wait(b, sr, tr, ss, sd):
            pltpu.make_async_copy(x_hbm.at[sidx.at[b]], sr, ss).wait()
            pltpu.make_async_copy(x_hbm.at[didx.at[b]], tr, sd).wait()

        def compute(b, sr, tr):
            @plsc.parallel_loop(0, BLK, step=1, unroll=8)
            def edge_body(j):
                acc = None
                for c in range(D // 32):
                    sw = plsc.bitcast(sr[j, pl.ds(c * 16, 16)], jnp.bfloat16)
                    tw = plsc.bitcast(tr[j, pl.ds(c * 16, 16)], jnp.bfloat16)
                    sa, sb = plsc.unpack(sw, format=plsc.PackFormat.INTERLEAVED)
                    ta, tb = plsc.unpack(tw, format=plsc.PackFormat.INTERLEAVED)
                    p = sa * ta + sb * tb
                    acc = p if acc is None else acc + p
                red[pl.ds(pl.multiple_of(j * 16, 16), 16)] = acc

            @plsc.parallel_loop(0, BLK // 16, step=1, unroll=1)
            def grp_body(g):
                gbase = g * 256 + rowbase
                outv = plsc.load_gather(red, [gbase])
                for c in range(1, 16):
                    outv = outv + plsc.load_gather(red, [gbase + c])
                outa[pl.ds(pl.multiple_of(b * BLK + g * 16, 16), 16)] = outv

        bufs = ((sr0, tr0, ss0, sd0), (sr1, tr1, ss1, sd1),
                (sr2, tr2, ss2, sd2), (sr3, tr3, ss3, sd3))
        ndeep = len(bufs)
        for k_ in range(ndeep):
            start(k_, *bufs[k_])

        def body(i, carry):
            for k_ in range(ndeep):
                b = i * ndeep + k_
                sr, tr, ss, sd = bufs[k_]
                wait(b, sr, tr, ss, sd)
                compute(b, sr, tr)

                @pl.when(b + ndeep < nb)
                def _():
                    start(b + ndeep, sr, tr, ss, sd)
            return carry

        lax.fori_loop(0, (nb - 1) // ndeep, body, 0)
        b_tail = ((nb - 1) // ndeep) * ndeep
        for k_ in range(nb - b_tail):
            sr, tr, ss, sd = bufs[k_]
            wait(b_tail + k_, sr, tr, ss, sd)
            compute(b_tail + k_, sr, tr)
        pltpu.sync_copy(outa, out_hbm.at[wid])

    return k(x, ei)


def kernel(x, edge_index, W1_l, b1_l, W1_r, W2_l, b2_l, W2_r):
    del W1_l, b1_l, W1_r, W2_l, b2_l, W2_r
    N, D = x.shape
    E = edge_index.shape[1]
    ei = edge_index.astype(jnp.int32).reshape(2, NW, (E // NW) // BLK, BLK)
    xi = jax.lax.bitcast_convert_type(
        x.astype(jnp.bfloat16).reshape(N, D // 2, 2), jnp.int32)
    out = _edge_dot(xi, ei, E, N, D)
    return out.reshape(E)
